# baseline (speedup 1.0000x reference)
import functools

import jax
import jax.numpy as jnp
from jax import lax
from jax.experimental import pallas as pl
from jax.experimental.pallas import tpu as pltpu
from jax.experimental.pallas import tpu_sc as plsc

B = 16384
G = 20
D = 64
LANES = 16
NC = 2
NS = 16
NW = NC * NS
ROWS_PER_W = B // NW
CHUNK = 32
NCHUNK = ROWS_PER_W // CHUNK
IDX_PER_CHUNK = CHUNK * G
STREAM_LEN = 128
NSTREAM = IDX_PER_CHUNK // STREAM_LEN


def _sc_body(center_hbm, ngram_hbm, win_hbm, wsub_hbm, out_hbm,
             cidx_v, nidx_v, srow_v, out_v, sem):
    wid = lax.axis_index("s") * NC + lax.axis_index("c")
    base_row = wid * ROWS_PER_W

    def chunk_body(i, carry):
        row0 = base_row + i * CHUNK
        pltpu.sync_copy(center_hbm.at[pl.ds(row0, CHUNK)], cidx_v)
        pltpu.sync_copy(ngram_hbm.at[pl.ds(row0 * G, IDX_PER_CHUNK)], nidx_v)
        copies = [pltpu.async_copy(win_hbm.at[cidx_v], out_v, sem)]
        for s in range(NSTREAM):
            copies.append(pltpu.async_copy(
                wsub_hbm.at[nidx_v.at[pl.ds(s * STREAM_LEN, STREAM_LEN)]],
                srow_v.at[pl.ds(s * STREAM_LEN, STREAM_LEN)], sem))
        for c in copies:
            c.wait()

        def row_body(r, c2):
            rg = r * G
            for d in range(D // LANES):
                sl = pl.ds(d * LANES, LANES)
                acc = out_v[r, sl]
                for g in range(G):
                    acc = acc + srow_v[rg + g, sl]
                out_v[r, sl] = acc
            return c2

        lax.fori_loop(0, CHUNK, row_body, 0)
        pltpu.sync_copy(out_v, out_hbm.at[pl.ds(row0, CHUNK)])
        return carry

    lax.fori_loop(0, NCHUNK, chunk_body, 0)


def kernel(center_ids, ngram_ids, W_in, W_sub):
    center_ids = center_ids.astype(jnp.int32)
    ngram_flat = ngram_ids.astype(jnp.int32).reshape(B * G)
    mesh = plsc.VectorSubcoreMesh(core_axis_name="c", subcore_axis_name="s")
    f = functools.partial(
        pl.kernel,
        mesh=mesh,
        compiler_params=pltpu.CompilerParams(use_tc_tiling_on_sc=False),
        out_type=jax.ShapeDtypeStruct((B, D), jnp.float32),
        scratch_types=[
            pltpu.VMEM((CHUNK,), jnp.int32),
            pltpu.VMEM((IDX_PER_CHUNK,), jnp.int32),
            pltpu.VMEM((IDX_PER_CHUNK, D), jnp.float32),
            pltpu.VMEM((CHUNK, D), jnp.float32),
            pltpu.SemaphoreType.DMA,
        ],
    )(_sc_body)
    return f(center_ids, ngram_flat, W_in, W_sub)
